# baseline (speedup 1.0000x reference)
import functools

import jax
import jax.numpy as jnp
from jax.experimental import pallas as pl
from jax.experimental.pallas import tpu as pltpu
from jax.experimental.pallas import tpu_sc as plsc

_LEVELS = 4
_K = 1024
_D = 256
_N = 16384
_T = 512
_SLABS = 2
_H = _N // _SLABS

_SC_INFO = plsc.get_sparse_core_info()
_NC, _NS = _SC_INFO.num_cores, _SC_INFO.num_subcores
_NW = _NC * _NS
_CHUNK = 128


def _argmin_body(r_ref, cb_ref, rnorm_ref, cnorm_ref, idx_ref):
    r = r_ref[...]
    cb = cb_ref[...]
    m = jax.lax.dot_general(r, cb, (((1,), (1,)), ((), ())),
                            preferred_element_type=jnp.float32)
    d2 = (rnorm_ref[...] - 2.0 * m) + cnorm_ref[...]
    dist = jnp.sqrt(jnp.maximum(d2, 0.0))
    idx_ref[...] = jnp.argmin(dist, axis=1).astype(jnp.int32).reshape(_T, 1)


def _tc_argmin(r, cb, rnorm, cnorm, row0):
    t0 = row0 // _T
    return pl.pallas_call(
        _argmin_body,
        grid=(_H // _T,),
        in_specs=[
            pl.BlockSpec((_T, _D), lambda i: (i + t0, 0)),
            pl.BlockSpec((_K, _D), lambda i: (0, 0)),
            pl.BlockSpec((_T, 1), lambda i: (i + t0, 0)),
            pl.BlockSpec((1, _K), lambda i: (0, 0)),
        ],
        out_specs=pl.BlockSpec((_T, 1), lambda i: (i, 0)),
        out_shape=jax.ShapeDtypeStruct((_H, 1), jnp.int32),
    )(r, cb, rnorm, cnorm)


def _sc_gather(cb, idx):
    n = idx.shape[0]
    b_per_w = n // _NW
    n_chunks = b_per_w // _CHUNK

    @functools.partial(
        pl.kernel,
        mesh=plsc.VectorSubcoreMesh(core_axis_name="c", subcore_axis_name="s"),
        out_type=jax.ShapeDtypeStruct((n, _D), jnp.float32),
        scratch_types=(
            [pltpu.VMEM((_CHUNK,), jnp.int32) for _ in range(n_chunks)]
            + [pltpu.VMEM((_CHUNK, _D), jnp.float32) for _ in range(n_chunks)]
            + [pltpu.SemaphoreType.DMA, pltpu.SemaphoreType.DMA]
        ),
    )
    def k(cb_hbm, idx_hbm, out_hbm, *bufs):
        idx_vs = bufs[:n_chunks]
        rows_vs = bufs[n_chunks:2 * n_chunks]
        gsem, osem = bufs[2 * n_chunks], bufs[2 * n_chunks + 1]
        wid = jax.lax.axis_index("s") * _NC + jax.lax.axis_index("c")
        bases = [wid * b_per_w + c * _CHUNK for c in range(n_chunks)]
        for c in range(n_chunks):
            pltpu.sync_copy(idx_hbm.at[pl.ds(bases[c], _CHUNK)], idx_vs[c])
        gathers = [pltpu.async_copy(cb_hbm.at[idx_vs[c]], rows_vs[c], gsem)
                   for c in range(n_chunks)]
        writes = []
        for c in range(n_chunks):
            gathers[c].wait()
            writes.append(pltpu.async_copy(
                rows_vs[c], out_hbm.at[pl.ds(bases[c], _CHUNK)], osem))
        for w in writes:
            w.wait()

    return k(cb, idx)


def _add_body(a_ref, b_ref, out_ref):
    out_ref[...] = a_ref[...] + b_ref[...]


def _tc_add(a, b):
    n = a.shape[0]
    spec = pl.BlockSpec((_T, _D), lambda i: (i, 0))
    return pl.pallas_call(
        _add_body,
        grid=(n // _T,),
        in_specs=[spec, spec],
        out_specs=spec,
        out_shape=jax.ShapeDtypeStruct((n, _D), jnp.float32),
    )(a, b)


def kernel(z, codebooks):
    cbs = [codebooks[l] for l in range(_LEVELS)]
    cnorms = [jnp.sum(cb * cb, axis=1).reshape(1, _K) for cb in cbs]
    rnorm0 = jnp.sum(z * z, axis=1, keepdims=True)
    qsum_slabs = []
    idx_slabs = []
    for s in range(_SLABS):
        row0 = s * _H
        r = None
        qsum = None
        idxs = []
        for l in range(_LEVELS):
            if l == 0:
                idx = _tc_argmin(z, cbs[0], rnorm0, cnorms[0], row0)
            else:
                rnorm = jnp.sum(r * r, axis=1, keepdims=True)
                idx = _tc_argmin(r, cbs[l], rnorm, cnorms[l], 0)
            q = _sc_gather(cbs[l], idx.reshape(_H))
            qsum = q if qsum is None else _tc_add(qsum, q)
            idxs.append(idx[:, 0])
            if l < _LEVELS - 1:
                r = (z[row0:row0 + _H] if l == 0 else r) - q
        qsum_slabs.append(qsum)
        idx_slabs.append(jnp.stack(idxs, axis=0))
    return (jnp.concatenate(qsum_slabs, axis=0),
            jnp.concatenate(idx_slabs, axis=1))


# speedup vs baseline: 1.1023x; 1.1023x over previous
import functools

import jax
import jax.numpy as jnp
from jax.experimental import pallas as pl
from jax.experimental.pallas import tpu as pltpu
from jax.experimental.pallas import tpu_sc as plsc

_LEVELS = 4
_K = 1024
_D = 256
_N = 16384
_T = 512
_SLABS = 2
_H = _N // _SLABS

_SC_INFO = plsc.get_sparse_core_info()
_NC, _NS = _SC_INFO.num_cores, _SC_INFO.num_subcores
_NW = _NC * _NS
_CHUNK = 128


def _argmin_body(r_ref, cb_ref, rnorm_ref, cnorm_ref, idx_ref):
    r = r_ref[...]
    cb = cb_ref[...]
    m = jax.lax.dot_general(r, cb, (((1,), (1,)), ((), ())),
                            preferred_element_type=jnp.float32)
    d2 = (rnorm_ref[...] - 2.0 * m) + cnorm_ref[...]
    dist = jnp.sqrt(jnp.maximum(d2, 0.0))
    idx_ref[...] = jnp.argmin(dist, axis=1).astype(jnp.int32).reshape(_T, 1)


def _tc_argmin(r, cb, rnorm, cnorm, row0):
    t0 = row0 // _T
    return pl.pallas_call(
        _argmin_body,
        grid=(_H // _T,),
        in_specs=[
            pl.BlockSpec((_T, _D), lambda i: (i + t0, 0)),
            pl.BlockSpec((_K, _D), lambda i: (0, 0)),
            pl.BlockSpec((_T, 1), lambda i: (i + t0, 0)),
            pl.BlockSpec((1, _K), lambda i: (0, 0)),
        ],
        out_specs=pl.BlockSpec((_T, 1), lambda i: (i, 0)),
        out_shape=jax.ShapeDtypeStruct((_H, 1), jnp.int32),
    )(r, cb, rnorm, cnorm)


def _sc_gather(cb, idx):
    n = idx.shape[0]
    b_per_w = n // _NW
    n_chunks = b_per_w // _CHUNK

    @functools.partial(
        pl.kernel,
        mesh=plsc.VectorSubcoreMesh(core_axis_name="c", subcore_axis_name="s"),
        out_type=jax.ShapeDtypeStruct((n, _D), jnp.float32),
        scratch_types=(
            [pltpu.VMEM((_CHUNK,), jnp.int32) for _ in range(n_chunks)]
            + [pltpu.VMEM((_CHUNK, _D), jnp.float32) for _ in range(n_chunks)]
            + [pltpu.SemaphoreType.DMA, pltpu.SemaphoreType.DMA]
        ),
    )
    def k(cb_hbm, idx_hbm, out_hbm, *bufs):
        idx_vs = bufs[:n_chunks]
        rows_vs = bufs[n_chunks:2 * n_chunks]
        gsem, osem = bufs[2 * n_chunks], bufs[2 * n_chunks + 1]
        wid = jax.lax.axis_index("s") * _NC + jax.lax.axis_index("c")
        bases = [wid * b_per_w + c * _CHUNK for c in range(n_chunks)]
        for c in range(n_chunks):
            pltpu.sync_copy(idx_hbm.at[pl.ds(bases[c], _CHUNK)], idx_vs[c])
        gathers = [pltpu.async_copy(cb_hbm.at[idx_vs[c]], rows_vs[c], gsem)
                   for c in range(n_chunks)]
        writes = []
        for c in range(n_chunks):
            gathers[c].wait()
            writes.append(pltpu.async_copy(
                rows_vs[c], out_hbm.at[pl.ds(bases[c], _CHUNK)], osem))
        for w in writes:
            w.wait()

    return k(cb, idx)


def _qsum_body(q0_ref, q1_ref, q2_ref, q3_ref, out_ref):
    out_ref[...] = ((q0_ref[...] + q1_ref[...]) + q2_ref[...]) + q3_ref[...]


def _tc_qsum(q0, q1, q2, q3):
    n = q0.shape[0]
    spec = pl.BlockSpec((_T, _D), lambda i: (i, 0))
    return pl.pallas_call(
        _qsum_body,
        grid=(n // _T,),
        in_specs=[spec, spec, spec, spec],
        out_specs=spec,
        out_shape=jax.ShapeDtypeStruct((n, _D), jnp.float32),
    )(q0, q1, q2, q3)


def kernel(z, codebooks):
    cbs = [codebooks[l] for l in range(_LEVELS)]
    cnorms = [jnp.sum(cb * cb, axis=1).reshape(1, _K) for cb in cbs]
    rnorm0 = jnp.sum(z * z, axis=1, keepdims=True)
    qsum_slabs = []
    idx_slabs = []
    for s in range(_SLABS):
        row0 = s * _H
        r = None
        qs = []
        idxs = []
        for l in range(_LEVELS):
            if l == 0:
                idx = _tc_argmin(z, cbs[0], rnorm0, cnorms[0], row0)
            else:
                rnorm = jnp.sum(r * r, axis=1, keepdims=True)
                idx = _tc_argmin(r, cbs[l], rnorm, cnorms[l], 0)
            q = _sc_gather(cbs[l], idx.reshape(_H))
            qs.append(q)
            idxs.append(idx[:, 0])
            if l < _LEVELS - 1:
                r = (z[row0:row0 + _H] if l == 0 else r) - q
        qsum_slabs.append(_tc_qsum(*qs))
        idx_slabs.append(jnp.stack(idxs, axis=0))
    return (jnp.concatenate(qsum_slabs, axis=0),
            jnp.concatenate(idx_slabs, axis=1))
